# baseline (speedup 1.0000x reference)
import functools

import jax
import jax.numpy as jnp
from jax import lax
from jax.experimental import pallas as pl
from jax.experimental.pallas import tpu as pltpu
from jax.experimental.pallas import tpu_sc as plsc

BATCH = 16384
D = 64
NC, NS = 2, 16
NW = NC * NS
BPW = BATCH // NW
CHUNK = 128
NCHUNK = BPW // CHUNK


def _sc_gather_all(heads, rels, tails, ent_h, ent_t, rel, rel_inv):
    mesh = plsc.VectorSubcoreMesh(core_axis_name="c", subcore_axis_name="s")
    row_ty = jax.ShapeDtypeStruct((BATCH, D), jnp.float32)

    @functools.partial(
        pl.kernel,
        out_type=(row_ty,) * 6,
        mesh=mesh,
        compiler_params=pltpu.CompilerParams(use_tc_tiling_on_sc=False),
        scratch_types=[
            pltpu.VMEM((BPW,), jnp.int32),
            pltpu.VMEM((BPW,), jnp.int32),
            pltpu.VMEM((BPW,), jnp.int32),
            pltpu.VMEM((BPW, D), jnp.float32),
            pltpu.SemaphoreType.DMA,
        ],
    )
    def k(heads_hbm, rels_hbm, tails_hbm, enth_hbm, entt_hbm, rel_hbm,
          relinv_hbm, hh_out, ht_out, th_out, tt_out, r_out, rinv_out,
          hidx, ridx, tidx, rows, sem):
        wid = lax.axis_index("s") * NC + lax.axis_index("c")
        base = wid * BPW
        pltpu.sync_copy(heads_hbm.at[pl.ds(base, BPW)], hidx)
        pltpu.sync_copy(rels_hbm.at[pl.ds(base, BPW)], ridx)
        pltpu.sync_copy(tails_hbm.at[pl.ds(base, BPW)], tidx)
        for table, idx, out in (
            (enth_hbm, hidx, hh_out),
            (enth_hbm, tidx, ht_out),
            (entt_hbm, hidx, th_out),
            (entt_hbm, tidx, tt_out),
            (rel_hbm, ridx, r_out),
            (relinv_hbm, ridx, rinv_out),
        ):
            copies = [
                pltpu.async_copy(
                    table.at[idx.at[pl.ds(c * CHUNK, CHUNK)]],
                    rows.at[pl.ds(c * CHUNK, CHUNK)],
                    sem,
                )
                for c in range(NCHUNK)
            ]
            for cp in copies:
                cp.wait()
            pltpu.sync_copy(rows, out.at[pl.ds(base, BPW)])

    return k(heads, rels, tails, ent_h, ent_t, rel, rel_inv)


def _tc_score(hh, ht, th, tt, r, rinv):
    blk = 2048

    def body(hh_ref, ht_ref, th_ref, tt_ref, r_ref, rinv_ref, o_ref):
        f = jnp.sum(hh_ref[...] * r_ref[...] * tt_ref[...], axis=1)
        inv = jnp.sum(ht_ref[...] * rinv_ref[...] * th_ref[...], axis=1)
        o_ref[...] = jnp.clip((f + inv) * 0.5, -20.0, 20.0)

    return pl.pallas_call(
        body,
        out_shape=jax.ShapeDtypeStruct((BATCH,), jnp.float32),
        grid=(BATCH // blk,),
        in_specs=[pl.BlockSpec((blk, D), lambda i: (i, 0))] * 6,
        out_specs=pl.BlockSpec((blk,), lambda i: (i,)),
    )(hh, ht, th, tt, r, rinv)


def kernel(heads, rels, tails, ent_h_embs, ent_t_embs, rel_embs, rel_inv_embs):
    heads = heads.astype(jnp.int32)
    rels = rels.astype(jnp.int32)
    tails = tails.astype(jnp.int32)
    hh, ht, th, tt, r, rinv = _sc_gather_all(
        heads, rels, tails, ent_h_embs, ent_t_embs, rel_embs, rel_inv_embs)
    return _tc_score(hh, ht, th, tt, r, rinv)
